# Initial kernel scaffold; baseline (speedup 1.0000x reference)
import jax
import jax.numpy as jnp
from jax.experimental import pallas as pl


def kernel(feature_map, rois):
    raise NotImplementedError("write your pallas kernel here")


# speedup vs baseline: 15.9804x; 15.9804x over previous
import jax
import jax.numpy as jnp
from jax import lax
from jax.experimental import pallas as pl
from jax.experimental.pallas import tpu as pltpu
from jax.experimental.pallas import tpu_sc as plsc

B, H, W, R = 8, 256, 256, 5000
NW = 32
RPW = 160
RPAD = NW * RPW

_F32 = jnp.float32
_I32 = jnp.int32


def _roi_body(fmap_hbm, rois_hbm, out_hbm, params_v, idx_v, buf_v, out_v, sem):
    wid = lax.axis_index("s") * 2 + lax.axis_index("c")
    pltpu.sync_copy(rois_hbm.at[wid], params_v)

    lane = lax.iota(_I32, 16)
    inv7 = _F32(1.0) / _F32(7.0)

    zero16 = lane * 0

    def roi_step(q, carry):
        qb = zero16 + q * 5
        b = plsc.load_gather(params_v, [qb]).astype(_I32)
        x1 = plsc.load_gather(params_v, [qb + 1])
        y1 = plsc.load_gather(params_v, [qb + 2])
        rw = plsc.load_gather(params_v, [qb + 3])
        rh = plsc.load_gather(params_v, [qb + 4])
        dx = (x1 + rw) - x1
        dy = (y1 + rh) - y1
        c0 = x1.astype(_I32) >> 4

        for half in range(2):
            t = (lane >> 1) + (8 * half)
            k = lane & 1
            j = jnp.minimum(t >> 1, 6)
            o = t & 1
            yv = y1 + dy * (j.astype(_F32) * inv7)
            y = yv.astype(_I32) + o
            cid = (b * 256 + y) * 16 + c0 + k
            idx_v[pl.ds(16 * half, 16)] = cid
        pltpu.async_copy(fmap_hbm.at[idx_v], buf_v, sem).wait()

        for v in range(4):
            m = lane + (16 * v)
            j = jnp.minimum(m // 7, 6)
            i = jnp.minimum(m - 7 * j, 6)
            jf = j.astype(_F32)
            i_f = i.astype(_F32)
            gx0 = x1 + dx * (i_f * inv7)
            gx1 = x1 + dx * ((i_f + 1.0) * inv7)
            gy0 = y1 + dy * (jf * inv7)
            gy1 = y1 + dy * ((jf + 1.0) * inv7)
            xl = gx0.astype(_I32)
            yt = gy0.astype(_I32)
            xr = gx1.astype(_I32)
            xr = jnp.where(gx1 > xr.astype(_F32), xr + 1, xr)
            yb = gy1.astype(_I32)
            yb = jnp.where(gy1 > yb.astype(_F32), yb + 1, yb)
            Xl = gx0 - xl.astype(_F32)
            Xr = xr.astype(_F32) - gx0
            Yt = gy0 - yt.astype(_F32)
            Yb = yb.astype(_F32) - gy0
            base = 64 * j + (xl - (c0 << 4))
            f00 = plsc.load_gather(buf_v, [base >> 4, base & 15])
            b01 = base + 1
            f01 = plsc.load_gather(buf_v, [b01 >> 4, b01 & 15])
            b10 = base + 32
            f10 = plsc.load_gather(buf_v, [b10 >> 4, b10 & 15])
            b11 = base + 33
            f11 = plsc.load_gather(buf_v, [b11 >> 4, b11 & 15])
            num = f00 * Xr * Yb + f01 * Xl * Yb + f10 * Xr * Yt + f11 * Xl * Yt
            den = (Xl + Xr) * (Yt + Yb)
            out_v[q, pl.ds(16 * v, 16)] = num / den
        return carry

    lax.fori_loop(0, RPW, roi_step, 0)
    pltpu.sync_copy(out_v, out_hbm.at[wid])


def _build_kernel():
    mesh = plsc.VectorSubcoreMesh(core_axis_name="c", subcore_axis_name="s")
    return pl.kernel(
        _roi_body,
        mesh=mesh,
        out_type=jax.ShapeDtypeStruct((NW, RPW, 64), _F32),
        scratch_types=[
            pltpu.VMEM((RPW * 5,), _F32),
            pltpu.VMEM((32,), _I32),
            pltpu.VMEM((32, 16), _F32),
            pltpu.VMEM((RPW, 64), _F32),
            pltpu.SemaphoreType.DMA,
        ],
        compiler_params=pltpu.CompilerParams(
            needs_layout_passes=False, use_tc_tiling_on_sc=False
        ),
    )


_sc_roi_align = _build_kernel()


def kernel(feature_map, rois):
    fm_chunks = feature_map.reshape(B * H * W // 16, 16)
    dummy = jnp.broadcast_to(
        jnp.array([0.0, 0.0, 0.0, 8.0, 8.0], _F32), (RPAD - R, 5)
    )
    rois_g = jnp.concatenate([rois, dummy], axis=0).reshape(NW, RPW * 5)
    out = _sc_roi_align(fm_chunks, rois_g)
    return out.reshape(RPAD, 64)[:R, :49].reshape(R, 7, 7)


# speedup vs baseline: 40.6443x; 2.5434x over previous
import jax
import jax.numpy as jnp
from jax import lax
from jax.experimental import pallas as pl
from jax.experimental.pallas import tpu as pltpu
from jax.experimental.pallas import tpu_sc as plsc

B, H, W, R = 8, 256, 256, 5000
NW = 32
RPW = 160
RPAD = NW * RPW
GS = 4
NG = RPW // GS

_F32 = jnp.float32
_I32 = jnp.int32


def _roi_body(fmap_hbm, rois_hbm, out_hbm, params_v, idx_v, buf_v, out_v, sem):
    wid = lax.axis_index("s") * 2 + lax.axis_index("c")
    pltpu.sync_copy(rois_hbm.at[wid], params_v)

    lane = lax.iota(_I32, 16)
    inv7 = _F32(1.0) / _F32(7.0)
    zero16 = lane * 0

    tif0, tif1, tjf0, tjf1, tj64 = [], [], [], [], []
    for v in range(4):
        m = lane + (16 * v)
        j = jnp.minimum(m // 7, 6)
        i = jnp.minimum(m - 7 * j, 6)
        tif0.append(i.astype(_F32) * inv7)
        tif1.append((i.astype(_F32) + 1.0) * inv7)
        tjf0.append(j.astype(_F32) * inv7)
        tjf1.append((j.astype(_F32) + 1.0) * inv7)
        tj64.append(j * 64)

    row_jf, row_o, row_k = [], [], []
    for half in range(2):
        t = (lane >> 1) + (8 * half)
        row_jf.append((t >> 1).astype(_F32) * inv7)
        row_o.append(t & 1)
        row_k.append(lane & 1)

    def params_of(q):
        qb = zero16 + q * 5
        b = plsc.load_gather(params_v, [qb]).astype(_I32)
        x1 = plsc.load_gather(params_v, [qb + 1])
        y1 = plsc.load_gather(params_v, [qb + 2])
        rw = plsc.load_gather(params_v, [qb + 3])
        rh = plsc.load_gather(params_v, [qb + 4])
        dx = (x1 + rw) - x1
        dy = (y1 + rh) - y1
        c0 = x1.astype(_I32) >> 4
        return b, x1, y1, dx, dy, c0

    def fire(g, carry):
        for r in range(GS):
            q = g * GS + r
            b, x1, y1, dx, dy, c0 = params_of(q)
            base = (b * 256) * 16 + c0
            for half in range(2):
                yv = y1 + dy * row_jf[half]
                y = yv.astype(_I32) + row_o[half]
                idx_v[g, pl.ds(r * 32 + 16 * half, 16)] = (
                    base + y * 16 + row_k[half]
                )
        pltpu.async_copy(
            fmap_hbm.at[idx_v.at[g]], buf_v.at[pl.ds(g * 128, 128)], sem
        )
        return carry

    lax.fori_loop(0, NG, fire, 0)

    pltpu.make_async_copy(fmap_hbm.at[pl.ds(0, RPW * 32)], buf_v, sem).wait()

    def compute_roi(q, carry):
        b, x1, y1, dx, dy, c0 = params_of(q)
        c016 = c0 << 4
        qbase = q * 512
        for v in range(4):
            gx0 = x1 + dx * tif0[v]
            gx1 = x1 + dx * tif1[v]
            gy0 = y1 + dy * tjf0[v]
            gy1 = y1 + dy * tjf1[v]
            xl = gx0.astype(_I32)
            yt = gy0.astype(_I32)
            xr = gx1.astype(_I32)
            xr = jnp.where(gx1 > xr.astype(_F32), xr + 1, xr)
            yb = gy1.astype(_I32)
            yb = jnp.where(gy1 > yb.astype(_F32), yb + 1, yb)
            xlf = xl.astype(_F32)
            xrf = xr.astype(_F32)
            ytf = yt.astype(_F32)
            ybf = yb.astype(_F32)
            Xl = gx0 - xlf
            Xr = xrf - gx0
            Yt = gy0 - ytf
            Yb = ybf - gy0
            b00 = (tj64[v] + qbase) + (xl - c016)
            b01 = b00 + 1
            b10 = b00 + 32
            b11 = b00 + 33
            f00 = plsc.load_gather(buf_v, [b00 >> 4, b00 & 15])
            f01 = plsc.load_gather(buf_v, [b01 >> 4, b01 & 15])
            f10 = plsc.load_gather(buf_v, [b10 >> 4, b10 & 15])
            f11 = plsc.load_gather(buf_v, [b11 >> 4, b11 & 15])
            XrYb = Xr * Yb
            XlYb = Xl * Yb
            XrYt = Xr * Yt
            XlYt = Xl * Yt
            num = f00 * XrYb + f01 * XlYb + f10 * XrYt + f11 * XlYt
            den = (Xl + Xr) * (Yt + Yb)
            out_v[q, pl.ds(16 * v, 16)] = num / den
        return carry

    lax.fori_loop(0, RPW, compute_roi, 0)
    pltpu.sync_copy(out_v, out_hbm.at[wid])


def _build_kernel():
    mesh = plsc.VectorSubcoreMesh(core_axis_name="c", subcore_axis_name="s")
    return pl.kernel(
        _roi_body,
        mesh=mesh,
        out_type=jax.ShapeDtypeStruct((NW, RPW, 64), _F32),
        scratch_types=[
            pltpu.VMEM((RPW * 5,), _F32),
            pltpu.VMEM((NG, GS * 32), _I32),
            pltpu.VMEM((RPW * 32, 16), _F32),
            pltpu.VMEM((RPW, 64), _F32),
            pltpu.SemaphoreType.DMA,
        ],
        compiler_params=pltpu.CompilerParams(
            needs_layout_passes=False, use_tc_tiling_on_sc=False
        ),
    )


_sc_roi_align = _build_kernel()


def kernel(feature_map, rois):
    fm_chunks = feature_map.reshape(B * H * W // 16, 16)
    dummy = jnp.broadcast_to(
        jnp.array([0.0, 0.0, 0.0, 8.0, 8.0], _F32), (RPAD - R, 5)
    )
    rois_g = jnp.concatenate([rois, dummy], axis=0).reshape(NW, RPW * 5)
    out = _sc_roi_align(fm_chunks, rois_g)
    return out.reshape(RPAD, 64)[:R, :49].reshape(R, 7, 7)


# speedup vs baseline: 41.2544x; 1.0150x over previous
import jax
import jax.numpy as jnp
from jax import lax
from jax.experimental import pallas as pl
from jax.experimental.pallas import tpu as pltpu
from jax.experimental.pallas import tpu_sc as plsc

B, H, W, R = 8, 256, 256, 5000
NW = 32
RPW = 160
RPAD = NW * RPW
GS = 4
NG = RPW // GS
NP = RPW // 16

_F32 = jnp.float32
_I32 = jnp.int32


def _roi_body(fmap_hbm, rois_hbm, out_hbm, params_v, XlS, XrS, YtS, YbS,
              colS, rowS, idx_v, buf_v, out_v, sem):
    wid = lax.axis_index("s") * 2 + lax.axis_index("c")
    pltpu.sync_copy(rois_hbm.at[wid], params_v)

    lane = lax.iota(_I32, 16)
    inv7 = _F32(1.0) / _F32(7.0)
    zero16 = lane * 0

    i_st, j_st, tj64 = [], [], []
    for v in range(4):
        m = lane + (16 * v)
        j = jnp.minimum(m // 7, 6)
        i = jnp.minimum(m - 7 * j, 6)
        i_st.append(i)
        j_st.append(j)
        tj64.append(j * 64)

    r8j_st, off_st = [], []
    for n in range(7):
        e = lane + (16 * n)
        r_e = e // 28
        rem = e - 28 * r_e
        t_e = rem >> 1
        r8j_st.append(r_e * 8 + (t_e >> 1))
        off_st.append((t_e & 1) * 16 + (rem & 1))

    def precompute(p, carry):
        qv = lane + p * 16
        pidx = qv * 5
        b = plsc.load_gather(params_v, [pidx]).astype(_I32)
        x1 = plsc.load_gather(params_v, [pidx + 1])
        y1 = plsc.load_gather(params_v, [pidx + 2])
        rw = plsc.load_gather(params_v, [pidx + 3])
        rh = plsc.load_gather(params_v, [pidx + 4])
        dx = (x1 + rw) - x1
        dy = (y1 + rh) - y1
        c0 = x1.astype(_I32) >> 4
        c016 = c0 << 4
        rowb = b * 4096 + c0
        sidx = qv * 8
        for i in range(7):
            ti = jnp.float32(i) * inv7
            ti1 = jnp.float32(i + 1) * inv7
            gx0 = x1 + dx * ti
            gx1 = x1 + dx * ti1
            xl = gx0.astype(_I32)
            xr = gx1.astype(_I32)
            xr = jnp.where(gx1 > xr.astype(_F32), xr + 1, xr)
            si = sidx + i
            plsc.store_scatter(XlS, [si], gx0 - xl.astype(_F32))
            plsc.store_scatter(XrS, [si], xr.astype(_F32) - gx0)
            plsc.store_scatter(colS, [si], xl - c016)
        for j in range(7):
            tj = jnp.float32(j) * inv7
            tj1 = jnp.float32(j + 1) * inv7
            gy0 = y1 + dy * tj
            gy1 = y1 + dy * tj1
            yt = gy0.astype(_I32)
            yb = gy1.astype(_I32)
            yb = jnp.where(gy1 > yb.astype(_F32), yb + 1, yb)
            sj = sidx + j
            plsc.store_scatter(YtS, [sj], gy0 - yt.astype(_F32))
            plsc.store_scatter(YbS, [sj], yb.astype(_F32) - gy0)
            plsc.store_scatter(rowS, [sj], rowb + (yt << 4))
        for gg in range(4):
            gid = p * 4 + gg
            sbase = zero16 + (p * 128 + gg * 32)
            for n in range(7):
                row = plsc.load_gather(rowS, [sbase + r8j_st[n]])
                idx_v[gid, pl.ds(16 * n, 16)] = row + off_st[n]
            pltpu.async_copy(
                fmap_hbm.at[idx_v.at[gid]],
                buf_v.at[pl.ds(gid * 112, 112)],
                sem,
            )
        return carry

    lax.fori_loop(0, NP, precompute, 0)

    pltpu.make_async_copy(fmap_hbm.at[pl.ds(0, RPW * 28)], buf_v, sem).wait()

    def compute_roi(q, carry):
        q8 = q * 8
        bb = zero16 + q * 448
        for v in range(4):
            xidx = q8 + i_st[v]
            yidx = q8 + j_st[v]
            col = plsc.load_gather(colS, [xidx])
            Xl = plsc.load_gather(XlS, [xidx])
            Xr = plsc.load_gather(XrS, [xidx])
            Yt = plsc.load_gather(YtS, [yidx])
            Yb = plsc.load_gather(YbS, [yidx])
            b00 = (bb + tj64[v]) + col
            b01 = b00 + 1
            b10 = b00 + 32
            b11 = b00 + 33
            f00 = plsc.load_gather(buf_v, [b00 >> 4, b00 & 15])
            f01 = plsc.load_gather(buf_v, [b01 >> 4, b01 & 15])
            f10 = plsc.load_gather(buf_v, [b10 >> 4, b10 & 15])
            f11 = plsc.load_gather(buf_v, [b11 >> 4, b11 & 15])
            XrYb = Xr * Yb
            XlYb = Xl * Yb
            XrYt = Xr * Yt
            XlYt = Xl * Yt
            num = f00 * XrYb + f01 * XlYb + f10 * XrYt + f11 * XlYt
            den = (Xl + Xr) * (Yt + Yb)
            out_v[q, pl.ds(16 * v, 16)] = num / den
        return carry

    lax.fori_loop(0, RPW, compute_roi, 0)
    pltpu.sync_copy(out_v, out_hbm.at[wid])


def _build_kernel():
    mesh = plsc.VectorSubcoreMesh(core_axis_name="c", subcore_axis_name="s")
    return pl.kernel(
        _roi_body,
        mesh=mesh,
        out_type=jax.ShapeDtypeStruct((NW, RPW, 64), _F32),
        scratch_types=[
            pltpu.VMEM((RPW * 5,), _F32),
            pltpu.VMEM((RPW * 8,), _F32),
            pltpu.VMEM((RPW * 8,), _F32),
            pltpu.VMEM((RPW * 8,), _F32),
            pltpu.VMEM((RPW * 8,), _F32),
            pltpu.VMEM((RPW * 8,), _I32),
            pltpu.VMEM((RPW * 8,), _I32),
            pltpu.VMEM((NG, GS * 28), _I32),
            pltpu.VMEM((RPW * 28, 16), _F32),
            pltpu.VMEM((RPW, 64), _F32),
            pltpu.SemaphoreType.DMA,
        ],
        compiler_params=pltpu.CompilerParams(
            needs_layout_passes=False, use_tc_tiling_on_sc=False
        ),
    )


_sc_roi_align = _build_kernel()


def kernel(feature_map, rois):
    fm_chunks = feature_map.reshape(B * H * W // 16, 16)
    dummy = jnp.broadcast_to(
        jnp.array([0.0, 0.0, 0.0, 8.0, 8.0], _F32), (RPAD - R, 5)
    )
    rois_g = jnp.concatenate([rois, dummy], axis=0).reshape(NW, RPW * 5)
    out = _sc_roi_align(fm_chunks, rois_g)
    return out.reshape(RPAD, 64)[:R, :49].reshape(R, 7, 7)
